# Initial kernel scaffold; baseline (speedup 1.0000x reference)
import jax
import jax.numpy as jnp
from jax.experimental import pallas as pl


def kernel(x, edge_index, W1a, b1a, W1b, b1b, W2a, b2a, W2b, b2b, bn_gamma, bn_beta, Wp, bp, bn2_gamma, bn2_beta, prelu_a):
    raise NotImplementedError("write your pallas kernel here")


# speedup vs baseline: 4.5662x; 4.5662x over previous
import math

import jax
import jax.numpy as jnp
from jax import lax
from jax.experimental import pallas as pl
from jax.experimental.pallas import tpu as pltpu
from jax.experimental.pallas import tpu_sc as plsc

NC = 2
NS = 16
NW = NC * NS
EDGE_CHUNK = 80
BN_SCALE = 1.0 / math.sqrt(1.0 + 1e-5)


def _scatter_body(z_hbm, src_hbm, dst_hbm, zero_hbm, out_hbm,
                  idx_s, idx_d, rows, acc, sem):
    cid = lax.axis_index("c")
    sid = lax.axis_index("s")
    n = z_hbm.shape[0]
    cp = ((n // NS + 7) // 8) * 8
    tail = n - cp * (NS - 1)
    r0 = sid * cp

    def _init_range(start, length):
        pltpu.sync_copy(zero_hbm.at[pl.ds(start, length)],
                        acc.at[pl.ds(start, length)])

    @pl.when(sid < NS - 1)
    def _():
        _init_range(r0, cp)

    @pl.when(sid == NS - 1)
    def _():
        _init_range(cp * (NS - 1), tail)

    plsc.subcore_barrier()

    e = src_hbm.shape[0]
    ept = e // NW
    wid = cid * NS + sid
    base = wid * ept

    def chunk(i, carry):
        off = base + i * EDGE_CHUNK
        pltpu.sync_copy(src_hbm.at[pl.ds(off, EDGE_CHUNK)], idx_s)
        pltpu.sync_copy(dst_hbm.at[pl.ds(off, EDGE_CHUNK)], idx_d)
        pltpu.async_copy(z_hbm.at[idx_s], rows, sem).wait()
        pltpu.sync_copy(rows, acc.at[idx_d], add=True)
        return carry

    lax.fori_loop(0, ept // EDGE_CHUNK, chunk, 0)
    plsc.subcore_barrier()

    def _pub_range(start, length):
        pltpu.sync_copy(acc.at[pl.ds(start, length)],
                        out_hbm.at[pl.ds(cid * n + start, length)])

    @pl.when(sid < NS - 1)
    def _():
        _pub_range(r0, cp)

    @pl.when(sid == NS - 1)
    def _():
        _pub_range(cp * (NS - 1), tail)


def _make_scatter(n, e, d, dtype):
    mesh = plsc.VectorSubcoreMesh(core_axis_name="c", subcore_axis_name="s",
                                  num_cores=NC, num_subcores=NS)
    return pl.kernel(
        _scatter_body,
        out_type=jax.ShapeDtypeStruct((NC * n, d), dtype),
        mesh=mesh,
        scratch_types=[
            pltpu.VMEM((EDGE_CHUNK,), jnp.int32),
            pltpu.VMEM((EDGE_CHUNK,), jnp.int32),
            pltpu.VMEM((EDGE_CHUNK, d), dtype),
            pltpu.VMEM_SHARED((n, d), dtype),
            pltpu.SemaphoreType.DMA,
        ],
    )


def _mlp_body(x_ref, p0_ref, p1_ref, wa_ref, ba_ref, wb_ref, bb_ref, o_ref):
    h = x_ref[...] + p0_ref[...] + p1_ref[...]
    h = jnp.maximum(
        jnp.dot(h, wa_ref[...], preferred_element_type=jnp.float32)
        + ba_ref[...], 0.0)
    h = jnp.dot(h, wb_ref[...], preferred_element_type=jnp.float32) + bb_ref[...]
    o_ref[...] = jnp.maximum(h, 0.0)


def _tail_body(x_ref, p0_ref, p1_ref, wa_ref, ba_ref, wb_ref, bb_ref,
               bng_ref, bnb_ref, wp_ref, bp_ref, bn2g_ref, bn2b_ref, pa_ref,
               z_ref, p_ref):
    h = x_ref[...] + p0_ref[...] + p1_ref[...]
    h = jnp.maximum(
        jnp.dot(h, wa_ref[...], preferred_element_type=jnp.float32)
        + ba_ref[...], 0.0)
    h = jnp.dot(h, wb_ref[...], preferred_element_type=jnp.float32) + bb_ref[...]
    h = jnp.maximum(h, 0.0)
    z = h * (bng_ref[...] * BN_SCALE) + bnb_ref[...]
    z_ref[...] = z
    q = jnp.dot(z, wp_ref[...], preferred_element_type=jnp.float32) + bp_ref[...]
    q = q * (bn2g_ref[...] * BN_SCALE) + bn2b_ref[...]
    p_ref[...] = jnp.where(q > 0, q, pa_ref[0, 0] * q)


_BLK = 2000


def _row_spec(d):
    return pl.BlockSpec((_BLK, d), lambda i: (i, 0))


def _rep_spec(r, c):
    return pl.BlockSpec((r, c), lambda i: (0, 0))


def _mlp_call(x, p0, p1, wa, ba, wb, bb):
    n, d = x.shape
    h = wa.shape[1]
    return pl.pallas_call(
        _mlp_body,
        grid=(n // _BLK,),
        in_specs=[_row_spec(d), _row_spec(d), _row_spec(d),
                  _rep_spec(d, h), _rep_spec(1, h),
                  _rep_spec(h, h), _rep_spec(1, h)],
        out_specs=_row_spec(h),
        out_shape=jax.ShapeDtypeStruct((n, h), x.dtype),
    )(x, p0, p1, wa, ba.reshape(1, h), wb, bb.reshape(1, h))


def _tail_call(x, p0, p1, wa, ba, wb, bb, bng, bnb, wp, bp, bn2g, bn2b, pa):
    n, d = x.shape
    h = wa.shape[1]
    vecs = [a.reshape(1, h) for a in (ba, bb, bng, bnb, bp, bn2g, bn2b)]
    return pl.pallas_call(
        _tail_body,
        grid=(n // _BLK,),
        in_specs=[_row_spec(d), _row_spec(d), _row_spec(d),
                  _rep_spec(d, h), _rep_spec(1, h),
                  _rep_spec(h, h), _rep_spec(1, h),
                  _rep_spec(1, h), _rep_spec(1, h),
                  _rep_spec(h, h), _rep_spec(1, h),
                  _rep_spec(1, h), _rep_spec(1, h),
                  pl.BlockSpec(memory_space=pltpu.SMEM)],
        out_specs=[_row_spec(h), _row_spec(h)],
        out_shape=[jax.ShapeDtypeStruct((n, h), x.dtype),
                   jax.ShapeDtypeStruct((n, h), x.dtype)],
    )(x, p0, p1, wa, vecs[0], wb, vecs[1], vecs[2], vecs[3], wp, vecs[4],
      vecs[5], vecs[6], pa.reshape(1, 1))


def kernel(x, edge_index, W1a, b1a, W1b, b1b, W2a, b2a, W2b, b2b,
           bn_gamma, bn_beta, Wp, bp, bn2_gamma, bn2_beta, prelu_a):
    n, d = x.shape
    e = edge_index.shape[1]
    src = edge_index[0]
    dst = edge_index[1]
    zeros = jnp.zeros_like(x)
    scat = _make_scatter(n, e, d, x.dtype)

    agg1 = scat(x, src, dst, zeros)
    z1 = _mlp_call(x, agg1[:n], agg1[n:], W1a, b1a, W1b, b1b)
    agg2 = scat(z1, src, dst, zeros)
    z, p = _tail_call(z1, agg2[:n], agg2[n:], W2a, b2a, W2b, b2b,
                      bn_gamma, bn_beta, Wp, bp, bn2_gamma, bn2_beta, prelu_a)
    return (z, p)


# speedup vs baseline: 9.5691x; 2.0956x over previous
import math

import jax
import jax.numpy as jnp
from jax import lax
from jax.experimental import pallas as pl
from jax.experimental.pallas import tpu as pltpu
from jax.experimental.pallas import tpu_sc as plsc

NC = 2
NS = 16
NW = NC * NS
EDGE_CHUNK = 128
CPT = 80
NBUF = 3
PAD_ROWS = 16
BN_SCALE = 1.0 / math.sqrt(1.0 + 1e-5)


def _scatter_body(z_hbm, src_hbm, dst_hbm, zero_hbm, out_hbm,
                  sb0, sb1, sb2, db0, db1, db2, rows, acc,
                  gsem, ssem, iss0, iss1, iss2, isd0, isd1, isd2):
    sbufs, dbufs = (sb0, sb1, sb2), (db0, db1, db2)
    isems_s, isems_d = (iss0, iss1, iss2), (isd0, isd1, isd2)
    cid = lax.axis_index("c")
    sid = lax.axis_index("s")
    n = z_hbm.shape[0]
    wid = cid * NS + sid
    c0 = wid * CPT

    def idx_start(j, slot):
        off = (c0 + j) * EDGE_CHUNK
        pltpu.async_copy(src_hbm.at[pl.ds(off, EDGE_CHUNK)], sbufs[slot],
                         isems_s[slot])
        pltpu.async_copy(dst_hbm.at[pl.ds(off, EDGE_CHUNK)], dbufs[slot],
                         isems_d[slot])

    def idx_wait(j, slot):
        off = (c0 + j) * EDGE_CHUNK
        pltpu.make_async_copy(src_hbm.at[pl.ds(off, EDGE_CHUNK)], sbufs[slot],
                              isems_s[slot]).wait()
        pltpu.make_async_copy(dst_hbm.at[pl.ds(off, EDGE_CHUNK)], dbufs[slot],
                              isems_d[slot]).wait()

    def gather_start(slot):
        pltpu.async_copy(z_hbm.at[sbufs[slot]], rows.at[slot], gsem)

    def gather_wait(slot):
        pltpu.make_async_copy(z_hbm.at[sbufs[slot]], rows.at[slot],
                              gsem).wait()

    def scat_start(slot):
        pltpu.async_copy(rows.at[slot], acc.at[dbufs[slot]], ssem, add=True)

    def scat_wait(slot):
        pltpu.make_async_copy(rows.at[slot], acc.at[dbufs[slot]], ssem).wait()

    for jj in range(3):
        idx_start(jj, jj)

    cp = ((n // NS + 7) // 8) * 8
    tail = n - cp * (NS - 1)
    r0 = sid * cp

    def _init_range(start, length):
        pltpu.sync_copy(zero_hbm.at[pl.ds(start, length)],
                        acc.at[pl.ds(start, length)])

    @pl.when(sid < NS - 1)
    def _():
        _init_range(r0, cp)

    @pl.when(sid == NS - 1)
    def _():
        _init_range(cp * (NS - 1), tail)

    idx_wait(0, 0)
    gather_start(0)
    plsc.subcore_barrier()

    def step(j, slot, do_scat_wait, do_idx_start, do_next):
        if do_scat_wait:
            scat_wait((slot + 2) % NBUF)
        if do_idx_start:
            idx_start(j + 2, (slot + 2) % NBUF)
        gather_wait(slot)
        scat_start(slot)
        if do_next:
            nslot = (slot + 1) % NBUF
            idx_wait(j + 1, nslot)
            gather_start(nslot)

    step(0, 0, do_scat_wait=False, do_idx_start=False, do_next=True)
    step(1, 1, do_scat_wait=True, do_idx_start=True, do_next=True)
    step(2, 2, do_scat_wait=True, do_idx_start=True, do_next=True)

    def body(k, carry):
        base = 3 + 3 * k
        for t in range(3):
            step(base + t, t, do_scat_wait=True, do_idx_start=True,
                 do_next=True)
        return carry

    lax.fori_loop(0, (CPT - 5) // 3, body, 0)

    step(CPT - 2, (CPT - 2) % NBUF, do_scat_wait=True, do_idx_start=False,
         do_next=True)
    step(CPT - 1, (CPT - 1) % NBUF, do_scat_wait=True, do_idx_start=False,
         do_next=False)
    scat_wait((CPT - 1) % NBUF)

    plsc.subcore_barrier()

    def _pub_range(start, length):
        pltpu.sync_copy(acc.at[pl.ds(start, length)],
                        out_hbm.at[pl.ds(cid * n + start, length)])

    @pl.when(sid < NS - 1)
    def _():
        _pub_range(r0, cp)

    @pl.when(sid == NS - 1)
    def _():
        _pub_range(cp * (NS - 1), tail)


def _make_scatter(n, d, dtype):
    mesh = plsc.VectorSubcoreMesh(core_axis_name="c", subcore_axis_name="s",
                                  num_cores=NC, num_subcores=NS)
    return pl.kernel(
        _scatter_body,
        out_type=jax.ShapeDtypeStruct((NC * n, d), dtype),
        mesh=mesh,
        scratch_types=(
            [pltpu.VMEM((EDGE_CHUNK,), jnp.int32) for _ in range(6)]
            + [pltpu.VMEM((NBUF, EDGE_CHUNK, d), dtype),
               pltpu.VMEM_SHARED((n + PAD_ROWS, d), dtype)]
            + [pltpu.SemaphoreType.DMA for _ in range(8)]
        ),
    )


def _mlp_body(x_ref, p0_ref, p1_ref, wa_ref, ba_ref, wb_ref, bb_ref, o_ref):
    h = x_ref[...] + p0_ref[...] + p1_ref[...]
    h = jnp.maximum(
        jnp.dot(h, wa_ref[...], preferred_element_type=jnp.float32)
        + ba_ref[...], 0.0)
    h = jnp.dot(h, wb_ref[...], preferred_element_type=jnp.float32) + bb_ref[...]
    o_ref[...] = jnp.maximum(h, 0.0)


def _tail_body(x_ref, p0_ref, p1_ref, wa_ref, ba_ref, wb_ref, bb_ref,
               bng_ref, bnb_ref, wp_ref, bp_ref, bn2g_ref, bn2b_ref, pa_ref,
               z_ref, p_ref):
    h = x_ref[...] + p0_ref[...] + p1_ref[...]
    h = jnp.maximum(
        jnp.dot(h, wa_ref[...], preferred_element_type=jnp.float32)
        + ba_ref[...], 0.0)
    h = jnp.dot(h, wb_ref[...], preferred_element_type=jnp.float32) + bb_ref[...]
    h = jnp.maximum(h, 0.0)
    z = h * (bng_ref[...] * BN_SCALE) + bnb_ref[...]
    z_ref[...] = z
    q = jnp.dot(z, wp_ref[...], preferred_element_type=jnp.float32) + bp_ref[...]
    q = q * (bn2g_ref[...] * BN_SCALE) + bn2b_ref[...]
    p_ref[...] = jnp.where(q > 0, q, pa_ref[0, 0] * q)


_BLK = 2000


def _row_spec(d):
    return pl.BlockSpec((_BLK, d), lambda i: (i, 0))


def _rep_spec(r, c):
    return pl.BlockSpec((r, c), lambda i: (0, 0))


def _mlp_call(x, p0, p1, wa, ba, wb, bb):
    n, d = x.shape
    h = wa.shape[1]
    return pl.pallas_call(
        _mlp_body,
        grid=(n // _BLK,),
        in_specs=[_row_spec(d), _row_spec(d), _row_spec(d),
                  _rep_spec(d, h), _rep_spec(1, h),
                  _rep_spec(h, h), _rep_spec(1, h)],
        out_specs=_row_spec(h),
        out_shape=jax.ShapeDtypeStruct((n, h), x.dtype),
    )(x, p0, p1, wa, ba.reshape(1, h), wb, bb.reshape(1, h))


def _tail_call(x, p0, p1, wa, ba, wb, bb, bng, bnb, wp, bp, bn2g, bn2b, pa):
    n, d = x.shape
    h = wa.shape[1]
    vecs = [a.reshape(1, h) for a in (ba, bb, bng, bnb, bp, bn2g, bn2b)]
    return pl.pallas_call(
        _tail_body,
        grid=(n // _BLK,),
        in_specs=[_row_spec(d), _row_spec(d), _row_spec(d),
                  _rep_spec(d, h), _rep_spec(1, h),
                  _rep_spec(h, h), _rep_spec(1, h),
                  _rep_spec(1, h), _rep_spec(1, h),
                  _rep_spec(h, h), _rep_spec(1, h),
                  _rep_spec(1, h), _rep_spec(1, h),
                  pl.BlockSpec(memory_space=pltpu.SMEM)],
        out_specs=[_row_spec(h), _row_spec(h)],
        out_shape=[jax.ShapeDtypeStruct((n, h), x.dtype),
                   jax.ShapeDtypeStruct((n, h), x.dtype)],
    )(x, p0, p1, wa, vecs[0], wb, vecs[1], vecs[2], vecs[3], wp, vecs[4],
      vecs[5], vecs[6], pa.reshape(1, 1))


def kernel(x, edge_index, W1a, b1a, W1b, b1b, W2a, b2a, W2b, b2b,
           bn_gamma, bn_beta, Wp, bp, bn2_gamma, bn2_beta, prelu_a):
    n, d = x.shape
    e = edge_index.shape[1]
    e_pad = NW * CPT * EDGE_CHUNK
    ar = jnp.arange(e_pad - e, dtype=jnp.int32)
    src = jnp.concatenate([edge_index[0], ar % n])
    dst = jnp.concatenate([edge_index[1], n + (ar % PAD_ROWS)])
    zeros = jnp.zeros_like(x)
    scat = _make_scatter(n, d, x.dtype)

    agg1 = scat(x, src, dst, zeros)
    z1 = _mlp_call(x, agg1[:n], agg1[n:], W1a, b1a, W1b, b1b)
    agg2 = scat(z1, src, dst, zeros)
    z, p = _tail_call(z1, agg2[:n], agg2[n:], W2a, b2a, W2b, b2b,
                      bn_gamma, bn_beta, Wp, bp, bn2_gamma, bn2_beta, prelu_a)
    return (z, p)


# speedup vs baseline: 11.8691x; 1.2404x over previous
import math

import jax
import jax.numpy as jnp
from jax import lax
from jax.experimental import pallas as pl
from jax.experimental.pallas import tpu as pltpu
from jax.experimental.pallas import tpu_sc as plsc

NC = 2
NS = 16
NW = NC * NS
EDGE_CHUNK = 128
CPT = 80
NBUF = 3
PAD_ROWS = 16
BN_SCALE = 1.0 / math.sqrt(1.0 + 1e-5)


def _scatter_body(z_hbm, src_hbm, dst_hbm, zero_hbm, out_hbm,
                  sb0, sb1, sb2, db0, db1, db2, db3, rows, acc,
                  gs0, gs1, ss0, ss1, ss2, ss3,
                  iss0, iss1, iss2, isd0, isd1, isd2, isd3):
    sbufs, dbufs = (sb0, sb1, sb2), (db0, db1, db2, db3)
    gsems, ssems = (gs0, gs1), (ss0, ss1, ss2, ss3)
    isems_s, isems_d = (iss0, iss1, iss2), (isd0, isd1, isd2, isd3)
    cid = lax.axis_index("c")
    sid = lax.axis_index("s")
    n = z_hbm.shape[0]
    wid = cid * NS + sid
    c0 = wid * CPT

    def idx_start(j, s3, s4):
        off = (c0 + j) * EDGE_CHUNK
        pltpu.async_copy(src_hbm.at[pl.ds(off, EDGE_CHUNK)], sbufs[s3],
                         isems_s[s3])
        pltpu.async_copy(dst_hbm.at[pl.ds(off, EDGE_CHUNK)], dbufs[s4],
                         isems_d[s4])

    def idx_wait(j, s3, s4):
        off = (c0 + j) * EDGE_CHUNK
        pltpu.make_async_copy(src_hbm.at[pl.ds(off, EDGE_CHUNK)], sbufs[s3],
                              isems_s[s3]).wait()
        pltpu.make_async_copy(dst_hbm.at[pl.ds(off, EDGE_CHUNK)], dbufs[s4],
                              isems_d[s4]).wait()

    def gather_start(s3, g2):
        pltpu.async_copy(z_hbm.at[sbufs[s3]], rows.at[s3], gsems[g2])

    def gather_wait(s3, g2):
        pltpu.make_async_copy(z_hbm.at[sbufs[s3]], rows.at[s3],
                              gsems[g2]).wait()

    def scat_start(s3, s4):
        pltpu.async_copy(rows.at[s3], acc.at[dbufs[s4]], ssems[s4], add=True)

    def scat_wait(s3, s4):
        pltpu.make_async_copy(rows.at[s3], acc.at[dbufs[s4]],
                              ssems[s4]).wait()

    idx_start(0, 0, 0)
    idx_start(1, 1, 1)

    cp = ((n // NS + 7) // 8) * 8
    tail = n - cp * (NS - 1)
    r0 = sid * cp

    def _init_range(start, length):
        pltpu.sync_copy(zero_hbm.at[pl.ds(start, length)],
                        acc.at[pl.ds(start, length)])

    @pl.when(sid < NS - 1)
    def _():
        _init_range(r0, cp)

    @pl.when(sid == NS - 1)
    def _():
        _init_range(cp * (NS - 1), tail)

    idx_wait(0, 0, 0)
    gather_start(0, 0)
    plsc.subcore_barrier()

    def step(j, s3, s4, g2, scat_w, idx_st, nxt):
        if scat_w:
            scat_wait((s3 + 1) % 3, (s4 + 2) % 4)
        if nxt:
            idx_wait(j + 1, (s3 + 1) % 3, (s4 + 1) % 4)
            gather_start((s3 + 1) % 3, (g2 + 1) % 2)
        if idx_st:
            idx_start(j + 2, (s3 + 2) % 3, (s4 + 2) % 4)
        gather_wait(s3, g2)
        scat_start(s3, s4)

    for jj in range(4):
        step(jj, jj % 3, jj % 4, jj % 2,
             scat_w=(jj >= 2), idx_st=True, nxt=True)

    def body(k, carry):
        base = 4 + 12 * k
        for t in range(12):
            step(base + t, (4 + t) % 3, t % 4, t % 2,
                 scat_w=True, idx_st=True, nxt=True)
        return carry

    lax.fori_loop(0, (CPT - 8) // 12, body, 0)

    for jj in range(CPT - 4, CPT):
        step(jj, jj % 3, jj % 4, jj % 2,
             scat_w=True, idx_st=(jj + 2 < CPT), nxt=(jj + 1 < CPT))
    scat_wait((CPT - 2) % 3, (CPT - 2) % 4)
    scat_wait((CPT - 1) % 3, (CPT - 1) % 4)

    plsc.subcore_barrier()

    def _pub_range(start, length):
        pltpu.sync_copy(acc.at[pl.ds(start, length)],
                        out_hbm.at[pl.ds(cid * n + start, length)])

    @pl.when(sid < NS - 1)
    def _():
        _pub_range(r0, cp)

    @pl.when(sid == NS - 1)
    def _():
        _pub_range(cp * (NS - 1), tail)


def _make_scatter(n, d, dtype):
    mesh = plsc.VectorSubcoreMesh(core_axis_name="c", subcore_axis_name="s",
                                  num_cores=NC, num_subcores=NS)
    return pl.kernel(
        _scatter_body,
        out_type=jax.ShapeDtypeStruct((NC * n, d), dtype),
        mesh=mesh,
        scratch_types=(
            [pltpu.VMEM((EDGE_CHUNK,), jnp.int32) for _ in range(7)]
            + [pltpu.VMEM((NBUF, EDGE_CHUNK, d), dtype),
               pltpu.VMEM_SHARED((n + PAD_ROWS, d), dtype)]
            + [pltpu.SemaphoreType.DMA for _ in range(13)]
        ),
    )


def _mlp_body(x_ref, p0_ref, p1_ref, wa_ref, ba_ref, wb_ref, bb_ref, o_ref):
    h = x_ref[...] + p0_ref[...] + p1_ref[...]
    h = jnp.maximum(
        jnp.dot(h, wa_ref[...], preferred_element_type=jnp.float32)
        + ba_ref[...], 0.0)
    h = jnp.dot(h, wb_ref[...], preferred_element_type=jnp.float32) + bb_ref[...]
    o_ref[...] = jnp.maximum(h, 0.0)


def _tail_body(x_ref, p0_ref, p1_ref, wa_ref, ba_ref, wb_ref, bb_ref,
               bng_ref, bnb_ref, wp_ref, bp_ref, bn2g_ref, bn2b_ref, pa_ref,
               z_ref, p_ref):
    h = x_ref[...] + p0_ref[...] + p1_ref[...]
    h = jnp.maximum(
        jnp.dot(h, wa_ref[...], preferred_element_type=jnp.float32)
        + ba_ref[...], 0.0)
    h = jnp.dot(h, wb_ref[...], preferred_element_type=jnp.float32) + bb_ref[...]
    h = jnp.maximum(h, 0.0)
    z = h * (bng_ref[...] * BN_SCALE) + bnb_ref[...]
    z_ref[...] = z
    q = jnp.dot(z, wp_ref[...], preferred_element_type=jnp.float32) + bp_ref[...]
    q = q * (bn2g_ref[...] * BN_SCALE) + bn2b_ref[...]
    p_ref[...] = jnp.where(q > 0, q, pa_ref[0, 0] * q)


_BLK = 2000


def _row_spec(d):
    return pl.BlockSpec((_BLK, d), lambda i: (i, 0))


def _rep_spec(r, c):
    return pl.BlockSpec((r, c), lambda i: (0, 0))


def _mlp_call(x, p0, p1, wa, ba, wb, bb):
    n, d = x.shape
    h = wa.shape[1]
    return pl.pallas_call(
        _mlp_body,
        grid=(n // _BLK,),
        in_specs=[_row_spec(d), _row_spec(d), _row_spec(d),
                  _rep_spec(d, h), _rep_spec(1, h),
                  _rep_spec(h, h), _rep_spec(1, h)],
        out_specs=_row_spec(h),
        out_shape=jax.ShapeDtypeStruct((n, h), x.dtype),
    )(x, p0, p1, wa, ba.reshape(1, h), wb, bb.reshape(1, h))


def _tail_call(x, p0, p1, wa, ba, wb, bb, bng, bnb, wp, bp, bn2g, bn2b, pa):
    n, d = x.shape
    h = wa.shape[1]
    vecs = [a.reshape(1, h) for a in (ba, bb, bng, bnb, bp, bn2g, bn2b)]
    return pl.pallas_call(
        _tail_body,
        grid=(n // _BLK,),
        in_specs=[_row_spec(d), _row_spec(d), _row_spec(d),
                  _rep_spec(d, h), _rep_spec(1, h),
                  _rep_spec(h, h), _rep_spec(1, h),
                  _rep_spec(1, h), _rep_spec(1, h),
                  _rep_spec(h, h), _rep_spec(1, h),
                  _rep_spec(1, h), _rep_spec(1, h),
                  pl.BlockSpec(memory_space=pltpu.SMEM)],
        out_specs=[_row_spec(h), _row_spec(h)],
        out_shape=[jax.ShapeDtypeStruct((n, h), x.dtype),
                   jax.ShapeDtypeStruct((n, h), x.dtype)],
    )(x, p0, p1, wa, vecs[0], wb, vecs[1], vecs[2], vecs[3], wp, vecs[4],
      vecs[5], vecs[6], pa.reshape(1, 1))


def kernel(x, edge_index, W1a, b1a, W1b, b1b, W2a, b2a, W2b, b2b,
           bn_gamma, bn_beta, Wp, bp, bn2_gamma, bn2_beta, prelu_a):
    n, d = x.shape
    e = edge_index.shape[1]
    e_pad = NW * CPT * EDGE_CHUNK
    ar = jnp.arange(e_pad - e, dtype=jnp.int32)
    src = jnp.concatenate([edge_index[0], ar % n])
    dst = jnp.concatenate([edge_index[1], n + (ar % PAD_ROWS)])
    zeros = jnp.zeros_like(x)
    scat = _make_scatter(n, d, x.dtype)

    agg1 = scat(x, src, dst, zeros)
    z1 = _mlp_call(x, agg1[:n], agg1[n:], W1a, b1a, W1b, b1b)
    agg2 = scat(z1, src, dst, zeros)
    z, p = _tail_call(z1, agg2[:n], agg2[n:], W2a, b2a, W2b, b2b,
                      bn_gamma, bn_beta, Wp, bp, bn2_gamma, bn2_beta, prelu_a)
    return (z, p)
